# Initial kernel scaffold; baseline (speedup 1.0000x reference)
import jax
import jax.numpy as jnp
from jax.experimental import pallas as pl


def kernel(node_logits, node_hidden, senders, receivers, edge_W0, edge_b0, edge_W1, edge_b1, edge_W2, edge_b2, node_W0, node_b0, node_W1, node_b1, node_W2, node_b2):
    raise NotImplementedError("write your pallas kernel here")


# speedup vs baseline: 26.1699x; 26.1699x over previous
import functools

import jax
import jax.numpy as jnp
from jax import lax
from jax.experimental import pallas as pl
from jax.experimental.pallas import tpu as pltpu
from jax.experimental.pallas import tpu_sc as plsc

N_NODES = 100000
N_EDGES = 3200000
TW = 16

NC = 2
NS = 16

GROUP = 125
GROUPS_PER_CHUNK = 8
CHUNK = GROUP * GROUPS_PER_CHUNK
N_CHUNKS = N_EDGES // CHUNK
CHUNKS_PER_TILE = N_CHUNKS // NS


def _edge_table_body(lg, hd, w0a, w0b, b0, w1, b1, w2l, b2l, w2h, b2h,
                     out_lo, out_hi):
    h = jnp.dot(lg[...], w0a[...], preferred_element_type=jnp.float32)
    h += jnp.dot(hd[...], w0b[...], preferred_element_type=jnp.float32)
    h = jnp.maximum(h + b0[...], 0.0)
    h = jnp.maximum(jnp.dot(h, w1[...], preferred_element_type=jnp.float32) + b1[...], 0.0)
    out_lo[...] = jnp.dot(h, w2l[...], preferred_element_type=jnp.float32) + b2l[...]
    out_hi[...] = jnp.dot(h, w2h[...], preferred_element_type=jnp.float32) + b2h[...]


def _edge_table(node_logits, node_hidden, eW0, eb0, eW1, eb1, eW2, eb2):
    blk = 5000
    grid = N_NODES // blk
    full = lambda shape: pl.BlockSpec(shape, lambda i: (0, 0))
    w2h = jnp.concatenate([eW2[:, 16:20], jnp.zeros((32, 12), jnp.float32)], axis=1)
    b2h = jnp.concatenate([eb2[16:20], jnp.zeros((12,), jnp.float32)])
    return pl.pallas_call(
        _edge_table_body,
        grid=(grid,),
        in_specs=[
            pl.BlockSpec((blk, 4), lambda i: (i, 0)),
            pl.BlockSpec((blk, 16), lambda i: (i, 0)),
            full((4, 64)), full((16, 64)), full((1, 64)),
            full((64, 32)), full((1, 32)),
            full((32, TW)), full((1, TW)),
            full((32, TW)), full((1, TW)),
        ],
        out_specs=[
            pl.BlockSpec((blk, TW), lambda i: (i, 0)),
            pl.BlockSpec((blk, TW), lambda i: (i, 0)),
        ],
        out_shape=[
            jax.ShapeDtypeStruct((N_NODES, TW), jnp.float32),
            jax.ShapeDtypeStruct((N_NODES, TW), jnp.float32),
        ],
    )(node_logits, node_hidden, eW0[:4], eW0[4:], eb0.reshape(1, 64),
      eW1, eb1.reshape(1, 32),
      eW2[:, :16], eb2[:16].reshape(1, TW),
      w2h, b2h.reshape(1, TW))


def _make_scatter():
    mesh = plsc.VectorSubcoreMesh(core_axis_name="c", subcore_axis_name="s")

    @functools.partial(
        pl.kernel,
        out_type=jax.ShapeDtypeStruct((NC, N_NODES, TW), jnp.float32),
        mesh=mesh,
        scratch_types=[
            pltpu.VMEM((GROUPS_PER_CHUNK, GROUP), jnp.int32),
            pltpu.VMEM((GROUPS_PER_CHUNK, GROUP), jnp.int32),
            pltpu.VMEM((CHUNK, TW), jnp.float32),
            pltpu.VMEM_SHARED((N_NODES, TW), jnp.float32),
            pltpu.SemaphoreType.DMA,
        ],
        compiler_params=pltpu.CompilerParams(use_tc_tiling_on_sc=False),
    )
    def scatter_kernel(tlo_hbm, thi_hbm, snd_hbm, rcv_hbm, zeros_hbm, out_hbm,
                       sbuf, rbuf, rows, acc, gsem):
        c = lax.axis_index("c")
        s = lax.axis_index("s")

        @pl.when(s == 0)
        def _():
            pltpu.sync_copy(zeros_hbm, acc)
        plsc.subcore_barrier()

        def run(tbl):
            def chunk(k, carry):
                base = s * CHUNKS_PER_TILE + k
                pltpu.sync_copy(snd_hbm.at[base], sbuf)
                pltpu.sync_copy(rcv_hbm.at[base], rbuf)
                descs = [
                    pltpu.async_copy(tbl.at[sbuf.at[j]],
                                     rows.at[pl.ds(j * GROUP, GROUP)], gsem)
                    for j in range(GROUPS_PER_CHUNK)
                ]
                for d in descs:
                    d.wait()
                for j in range(GROUPS_PER_CHUNK):
                    pltpu.sync_copy(rows.at[pl.ds(j * GROUP, GROUP)],
                                    acc.at[rbuf.at[j]], add=True)
                return carry

            lax.fori_loop(0, CHUNKS_PER_TILE, chunk, 0)

        @pl.when(c == 0)
        def _():
            run(tlo_hbm)

        @pl.when(c == 1)
        def _():
            run(thi_hbm)

        plsc.subcore_barrier()

        @pl.when(s == 0)
        def _():
            pltpu.sync_copy(acc, out_hbm.at[c])

    return scatter_kernel


_scatter = _make_scatter()


def _node_body(lg, hd, parts, w0a, w0b, w0c, w0d, b0, w1, b1, w2l, b2l,
               w2h, b2h, out_lg, out_hd):
    h = jnp.dot(lg[...], w0a[...], preferred_element_type=jnp.float32)
    h += jnp.dot(hd[...], w0b[...], preferred_element_type=jnp.float32)
    h += jnp.dot(parts[0], w0c[...], preferred_element_type=jnp.float32)
    h += jnp.dot(parts[1], w0d[...], preferred_element_type=jnp.float32)
    h = jnp.maximum(h + b0[...], 0.0)
    h = jnp.maximum(jnp.dot(h, w1[...], preferred_element_type=jnp.float32) + b1[...], 0.0)
    out_lg[...] = jnp.dot(h, w2l[...], preferred_element_type=jnp.float32) + b2l[...]
    out_hd[...] = jnp.dot(h, w2h[...], preferred_element_type=jnp.float32) + b2h[...]


def _node_update(node_logits, node_hidden, parts, nW0, nb0, nW1, nb1, nW2, nb2):
    blk = 5000
    grid = N_NODES // blk
    full = lambda shape: pl.BlockSpec(shape, lambda i: tuple(0 for _ in shape))
    w0c = nW0[20:36]
    w0d = jnp.concatenate([nW0[36:40], jnp.zeros((12, 64), jnp.float32)], axis=0)
    return pl.pallas_call(
        _node_body,
        grid=(grid,),
        in_specs=[
            pl.BlockSpec((blk, 4), lambda i: (i, 0)),
            pl.BlockSpec((blk, 16), lambda i: (i, 0)),
            pl.BlockSpec((NC, blk, TW), lambda i: (0, i, 0)),
            full((4, 64)), full((16, 64)), full((TW, 64)), full((TW, 64)),
            full((1, 64)),
            full((64, 32)), full((1, 32)),
            full((32, 4)), full((1, 4)),
            full((32, 16)), full((1, 16)),
        ],
        out_specs=[
            pl.BlockSpec((blk, 4), lambda i: (i, 0)),
            pl.BlockSpec((blk, 16), lambda i: (i, 0)),
        ],
        out_shape=[
            jax.ShapeDtypeStruct((N_NODES, 4), jnp.float32),
            jax.ShapeDtypeStruct((N_NODES, 16), jnp.float32),
        ],
    )(node_logits, node_hidden, parts,
      nW0[:4], nW0[4:20], w0c, w0d,
      nb0.reshape(1, 64),
      nW1, nb1.reshape(1, 32),
      nW2[:, :4], nb2[:4].reshape(1, 4), nW2[:, 4:], nb2[4:].reshape(1, 16))


def kernel(node_logits, node_hidden, senders, receivers,
           edge_W0, edge_b0, edge_W1, edge_b1, edge_W2, edge_b2,
           node_W0, node_b0, node_W1, node_b1, node_W2, node_b2):
    t_lo, t_hi = _edge_table(node_logits, node_hidden,
                             edge_W0, edge_b0, edge_W1, edge_b1,
                             edge_W2, edge_b2)
    snd = senders.reshape(N_CHUNKS, GROUPS_PER_CHUNK, GROUP)
    rcv = receivers.reshape(N_CHUNKS, GROUPS_PER_CHUNK, GROUP)
    zeros = jnp.zeros((N_NODES, TW), jnp.float32)
    parts = _scatter(t_lo, t_hi, snd, rcv, zeros)
    new_logits, new_hidden = _node_update(
        node_logits, node_hidden, parts,
        node_W0, node_b0, node_W1, node_b1, node_W2, node_b2)
    return (new_logits, new_hidden)


# speedup vs baseline: 35.1859x; 1.3445x over previous
import functools

import jax
import jax.numpy as jnp
from jax import lax
from jax.experimental import pallas as pl
from jax.experimental.pallas import tpu as pltpu
from jax.experimental.pallas import tpu_sc as plsc

N_NODES = 100000
N_EDGES = 3200000
TW = 16

NC = 2
NS = 16

GROUP = 125
GROUPS_PER_CHUNK = 4
CHUNK = GROUP * GROUPS_PER_CHUNK
N_CHUNKS = N_EDGES // CHUNK
CHUNKS_PER_TILE = N_CHUNKS // NS
BODIES = CHUNKS_PER_TILE // 2


def _edge_table_body(lg, hd, w0a, w0b, b0, w1, b1, w2l, b2l, w2h, b2h,
                     out_lo, out_hi):
    h = jnp.dot(lg[...], w0a[...], preferred_element_type=jnp.float32)
    h += jnp.dot(hd[...], w0b[...], preferred_element_type=jnp.float32)
    h = jnp.maximum(h + b0[...], 0.0)
    h = jnp.maximum(jnp.dot(h, w1[...], preferred_element_type=jnp.float32) + b1[...], 0.0)
    out_lo[...] = jnp.dot(h, w2l[...], preferred_element_type=jnp.float32) + b2l[...]
    out_hi[...] = jnp.dot(h, w2h[...], preferred_element_type=jnp.float32) + b2h[...]


def _edge_table(node_logits, node_hidden, eW0, eb0, eW1, eb1, eW2, eb2):
    blk = 5000
    grid = N_NODES // blk
    full = lambda shape: pl.BlockSpec(shape, lambda i: (0, 0))
    w2h = jnp.concatenate([eW2[:, 16:20], jnp.zeros((32, 12), jnp.float32)], axis=1)
    b2h = jnp.concatenate([eb2[16:20], jnp.zeros((12,), jnp.float32)])
    return pl.pallas_call(
        _edge_table_body,
        grid=(grid,),
        in_specs=[
            pl.BlockSpec((blk, 4), lambda i: (i, 0)),
            pl.BlockSpec((blk, 16), lambda i: (i, 0)),
            full((4, 64)), full((16, 64)), full((1, 64)),
            full((64, 32)), full((1, 32)),
            full((32, TW)), full((1, TW)),
            full((32, TW)), full((1, TW)),
        ],
        out_specs=[
            pl.BlockSpec((blk, TW), lambda i: (i, 0)),
            pl.BlockSpec((blk, TW), lambda i: (i, 0)),
        ],
        out_shape=[
            jax.ShapeDtypeStruct((N_NODES, TW), jnp.float32),
            jax.ShapeDtypeStruct((N_NODES, TW), jnp.float32),
        ],
    )(node_logits, node_hidden, eW0[:4], eW0[4:], eb0.reshape(1, 64),
      eW1, eb1.reshape(1, 32),
      eW2[:, :16], eb2[:16].reshape(1, TW),
      w2h, b2h.reshape(1, TW))


def _make_scatter():
    mesh = plsc.VectorSubcoreMesh(core_axis_name="c", subcore_axis_name="s")

    @functools.partial(
        pl.kernel,
        out_type=jax.ShapeDtypeStruct((NC, N_NODES, TW), jnp.float32),
        mesh=mesh,
        scratch_types=[
            pltpu.VMEM((GROUPS_PER_CHUNK, GROUP), jnp.int32),
            pltpu.VMEM((GROUPS_PER_CHUNK, GROUP), jnp.int32),
            pltpu.VMEM((GROUPS_PER_CHUNK, GROUP), jnp.int32),
            pltpu.VMEM((GROUPS_PER_CHUNK, GROUP), jnp.int32),
            pltpu.VMEM((CHUNK, TW), jnp.float32),
            pltpu.VMEM((CHUNK, TW), jnp.float32),
            pltpu.VMEM_SHARED((N_NODES, TW), jnp.float32),
            pltpu.SemaphoreType.DMA,
            pltpu.SemaphoreType.DMA,
            pltpu.SemaphoreType.DMA,
            pltpu.SemaphoreType.DMA,
            pltpu.SemaphoreType.DMA,
            pltpu.SemaphoreType.DMA,
        ],
        compiler_params=pltpu.CompilerParams(use_tc_tiling_on_sc=False),
    )
    def scatter_kernel(tlo_hbm, thi_hbm, snd_hbm, rcv_hbm, zeros_hbm, out_hbm,
                       sbufA, rbufA, sbufB, rbufB, rowsA, rowsB, acc,
                       gsemA, gsemB, ssemA, ssemB, isemA, isemB):
        c = lax.axis_index("c")
        s = lax.axis_index("s")

        @pl.when(s == 0)
        def _():
            pltpu.sync_copy(zeros_hbm, acc)
        plsc.subcore_barrier()

        def run(tbl):
            def fire_idx(k, sb, rb, isem):
                base = s * CHUNKS_PER_TILE + k
                pltpu.async_copy(snd_hbm.at[base], sb, isem)
                pltpu.async_copy(rcv_hbm.at[base], rb, isem)

            def drain_idx(sb, rb, isem):
                pltpu.make_async_copy(snd_hbm.at[0], sb, isem).wait()
                pltpu.make_async_copy(rcv_hbm.at[0], rb, isem).wait()

            def fire_gather(sb, rows, gsem):
                for j in range(GROUPS_PER_CHUNK):
                    pltpu.async_copy(tbl.at[sb.at[j]],
                                     rows.at[pl.ds(j * GROUP, GROUP)], gsem)

            def drain_gather(sb, rows, gsem):
                for j in range(GROUPS_PER_CHUNK):
                    pltpu.make_async_copy(tbl.at[sb.at[j]],
                                          rows.at[pl.ds(j * GROUP, GROUP)],
                                          gsem).wait()

            def fire_scatter(rb, rows, ssem):
                for j in range(GROUPS_PER_CHUNK):
                    pltpu.async_copy(rows.at[pl.ds(j * GROUP, GROUP)],
                                     acc.at[rb.at[j]], ssem, add=True)

            def drain_scatter(rb, rows, ssem):
                for j in range(GROUPS_PER_CHUNK):
                    pltpu.make_async_copy(rows.at[pl.ds(j * GROUP, GROUP)],
                                          acc.at[rb.at[j]], ssem).wait()

            pltpu.sync_copy(zeros_hbm.at[pl.ds(0, CHUNK)], rowsB)
            pltpu.sync_copy(rcv_hbm.at[s * CHUNKS_PER_TILE], rbufB)
            fire_scatter(rbufB, rowsB, ssemB)
            fire_idx(0, sbufA, rbufA, isemA)
            drain_idx(sbufA, rbufA, isemA)
            fire_gather(sbufA, rowsA, gsemA)

            def body(t, carry):
                k1 = 2 * t + 1
                k2 = lax.rem(2 * t + 2, CHUNKS_PER_TILE)
                drain_scatter(rbufB, rowsB, ssemB)
                fire_idx(k1, sbufB, rbufB, isemB)
                drain_idx(sbufB, rbufB, isemB)
                fire_gather(sbufB, rowsB, gsemB)
                drain_gather(sbufA, rowsA, gsemA)
                fire_scatter(rbufA, rowsA, ssemA)
                drain_scatter(rbufA, rowsA, ssemA)
                fire_idx(k2, sbufA, rbufA, isemA)
                drain_idx(sbufA, rbufA, isemA)
                fire_gather(sbufA, rowsA, gsemA)
                drain_gather(sbufB, rowsB, gsemB)
                fire_scatter(rbufB, rowsB, ssemB)
                return carry

            lax.fori_loop(0, BODIES, body, 0)

            drain_gather(sbufA, rowsA, gsemA)
            drain_scatter(rbufB, rowsB, ssemB)

        @pl.when(c == 0)
        def _():
            run(tlo_hbm)

        @pl.when(c == 1)
        def _():
            run(thi_hbm)

        plsc.subcore_barrier()

        @pl.when(s == 0)
        def _():
            pltpu.sync_copy(acc, out_hbm.at[c])

    return scatter_kernel


_scatter = _make_scatter()


def _node_body(lg, hd, parts, w0a, w0b, w0c, w0d, b0, w1, b1, w2l, b2l,
               w2h, b2h, out_lg, out_hd):
    h = jnp.dot(lg[...], w0a[...], preferred_element_type=jnp.float32)
    h += jnp.dot(hd[...], w0b[...], preferred_element_type=jnp.float32)
    h += jnp.dot(parts[0], w0c[...], preferred_element_type=jnp.float32)
    h += jnp.dot(parts[1], w0d[...], preferred_element_type=jnp.float32)
    h = jnp.maximum(h + b0[...], 0.0)
    h = jnp.maximum(jnp.dot(h, w1[...], preferred_element_type=jnp.float32) + b1[...], 0.0)
    out_lg[...] = jnp.dot(h, w2l[...], preferred_element_type=jnp.float32) + b2l[...]
    out_hd[...] = jnp.dot(h, w2h[...], preferred_element_type=jnp.float32) + b2h[...]


def _node_update(node_logits, node_hidden, parts, nW0, nb0, nW1, nb1, nW2, nb2):
    blk = 5000
    grid = N_NODES // blk
    full = lambda shape: pl.BlockSpec(shape, lambda i: tuple(0 for _ in shape))
    w0c = nW0[20:36]
    w0d = jnp.concatenate([nW0[36:40], jnp.zeros((12, 64), jnp.float32)], axis=0)
    return pl.pallas_call(
        _node_body,
        grid=(grid,),
        in_specs=[
            pl.BlockSpec((blk, 4), lambda i: (i, 0)),
            pl.BlockSpec((blk, 16), lambda i: (i, 0)),
            pl.BlockSpec((NC, blk, TW), lambda i: (0, i, 0)),
            full((4, 64)), full((16, 64)), full((TW, 64)), full((TW, 64)),
            full((1, 64)),
            full((64, 32)), full((1, 32)),
            full((32, 4)), full((1, 4)),
            full((32, 16)), full((1, 16)),
        ],
        out_specs=[
            pl.BlockSpec((blk, 4), lambda i: (i, 0)),
            pl.BlockSpec((blk, 16), lambda i: (i, 0)),
        ],
        out_shape=[
            jax.ShapeDtypeStruct((N_NODES, 4), jnp.float32),
            jax.ShapeDtypeStruct((N_NODES, 16), jnp.float32),
        ],
    )(node_logits, node_hidden, parts,
      nW0[:4], nW0[4:20], w0c, w0d,
      nb0.reshape(1, 64),
      nW1, nb1.reshape(1, 32),
      nW2[:, :4], nb2[:4].reshape(1, 4), nW2[:, 4:], nb2[4:].reshape(1, 16))


def kernel(node_logits, node_hidden, senders, receivers,
           edge_W0, edge_b0, edge_W1, edge_b1, edge_W2, edge_b2,
           node_W0, node_b0, node_W1, node_b1, node_W2, node_b2):
    t_lo, t_hi = _edge_table(node_logits, node_hidden,
                             edge_W0, edge_b0, edge_W1, edge_b1,
                             edge_W2, edge_b2)
    snd = senders.reshape(N_CHUNKS, GROUPS_PER_CHUNK, GROUP)
    rcv = receivers.reshape(N_CHUNKS, GROUPS_PER_CHUNK, GROUP)
    zeros = jnp.zeros((N_NODES, TW), jnp.float32)
    parts = _scatter(t_lo, t_hi, snd, rcv, zeros)
    new_logits, new_hidden = _node_update(
        node_logits, node_hidden, parts,
        node_W0, node_b0, node_W1, node_b1, node_W2, node_b2)
    return (new_logits, new_hidden)


# speedup vs baseline: 39.5749x; 1.1247x over previous
import functools

import jax
import jax.numpy as jnp
from jax import lax
from jax.experimental import pallas as pl
from jax.experimental.pallas import tpu as pltpu
from jax.experimental.pallas import tpu_sc as plsc

N_NODES = 100000
N_EDGES = 3200000
TW = 16

NC = 2
NS = 16

GROUP = 80
GROUPS_PER_CHUNK = 5
CHUNK = GROUP * GROUPS_PER_CHUNK
N_CHUNKS = N_EDGES // CHUNK
CHUNKS_PER_TILE = N_CHUNKS // NS
BODIES = CHUNKS_PER_TILE // 2


def _edge_table_body(lg, hd, w0a, w0b, b0, w1, b1, w2l, b2l, w2h, b2h,
                     out_lo, out_hi):
    h = jnp.dot(lg[...], w0a[...], preferred_element_type=jnp.float32)
    h += jnp.dot(hd[...], w0b[...], preferred_element_type=jnp.float32)
    h = jnp.maximum(h + b0[...], 0.0)
    h = jnp.maximum(jnp.dot(h, w1[...], preferred_element_type=jnp.float32) + b1[...], 0.0)
    out_lo[...] = jnp.dot(h, w2l[...], preferred_element_type=jnp.float32) + b2l[...]
    out_hi[...] = jnp.dot(h, w2h[...], preferred_element_type=jnp.float32) + b2h[...]


def _edge_table(node_logits, node_hidden, eW0, eb0, eW1, eb1, eW2, eb2):
    blk = 5000
    grid = N_NODES // blk
    full = lambda shape: pl.BlockSpec(shape, lambda i: (0, 0))
    w2h = jnp.concatenate([eW2[:, 16:20], jnp.zeros((32, 12), jnp.float32)], axis=1)
    b2h = jnp.concatenate([eb2[16:20], jnp.zeros((12,), jnp.float32)])
    return pl.pallas_call(
        _edge_table_body,
        grid=(grid,),
        in_specs=[
            pl.BlockSpec((blk, 4), lambda i: (i, 0)),
            pl.BlockSpec((blk, 16), lambda i: (i, 0)),
            full((4, 64)), full((16, 64)), full((1, 64)),
            full((64, 32)), full((1, 32)),
            full((32, TW)), full((1, TW)),
            full((32, TW)), full((1, TW)),
        ],
        out_specs=[
            pl.BlockSpec((blk, TW), lambda i: (i, 0)),
            pl.BlockSpec((blk, TW), lambda i: (i, 0)),
        ],
        out_shape=[
            jax.ShapeDtypeStruct((N_NODES, TW), jnp.float32),
            jax.ShapeDtypeStruct((N_NODES, TW), jnp.float32),
        ],
    )(node_logits, node_hidden, eW0[:4], eW0[4:], eb0.reshape(1, 64),
      eW1, eb1.reshape(1, 32),
      eW2[:, :16], eb2[:16].reshape(1, TW),
      w2h, b2h.reshape(1, TW))


def _make_scatter():
    mesh = plsc.VectorSubcoreMesh(core_axis_name="c", subcore_axis_name="s")

    @functools.partial(
        pl.kernel,
        out_type=jax.ShapeDtypeStruct((NC, N_NODES, TW), jnp.float32),
        mesh=mesh,
        scratch_types=[
            pltpu.VMEM((2 * CHUNK,), jnp.int32),
            pltpu.VMEM((2 * CHUNK,), jnp.int32),
            pltpu.VMEM((2 * CHUNK,), jnp.int32),
            pltpu.VMEM((2 * CHUNK,), jnp.int32),
            pltpu.VMEM((CHUNK, TW), jnp.float32),
            pltpu.VMEM((CHUNK, TW), jnp.float32),
            pltpu.VMEM_SHARED((N_NODES, TW), jnp.float32),
            pltpu.SemaphoreType.DMA,
            pltpu.SemaphoreType.DMA,
            pltpu.SemaphoreType.DMA,
            pltpu.SemaphoreType.DMA,
            pltpu.SemaphoreType.DMA,
        ],
        compiler_params=pltpu.CompilerParams(use_tc_tiling_on_sc=False),
    )
    def scatter_kernel(tlo_hbm, thi_hbm, snd_hbm, rcv_hbm, zeros_hbm, out_hbm,
                       sbufP, rbufP, sbufQ, rbufQ, rowsA, rowsB, acc,
                       gsemA, gsemB, ssemA, ssemB, isem):
        c = lax.axis_index("c")
        s = lax.axis_index("s")

        @pl.when(s == 0)
        def _():
            pltpu.sync_copy(zeros_hbm, acc)
        plsc.subcore_barrier()

        def run(tbl):
            def fire_idxpair(pair, sb, rb):
                base = (s * CHUNKS_PER_TILE + 2 * pair) * CHUNK
                pltpu.async_copy(snd_hbm.at[pl.ds(base, 2 * CHUNK)], sb, isem)
                pltpu.async_copy(rcv_hbm.at[pl.ds(base, 2 * CHUNK)], rb, isem)

            def drain_idxpair(sb, rb):
                pltpu.make_async_copy(snd_hbm.at[pl.ds(0, 2 * CHUNK)], sb,
                                      isem).wait()
                pltpu.make_async_copy(rcv_hbm.at[pl.ds(0, 2 * CHUNK)], rb,
                                      isem).wait()

            def fire_gather(sb, half, rows, gsem):
                for j in range(GROUPS_PER_CHUNK):
                    pltpu.async_copy(
                        tbl.at[sb.at[pl.ds(half * CHUNK + j * GROUP, GROUP)]],
                        rows.at[pl.ds(j * GROUP, GROUP)], gsem)

            def drain_gather(sb, half, rows, gsem):
                for j in range(GROUPS_PER_CHUNK):
                    pltpu.make_async_copy(
                        tbl.at[sb.at[pl.ds(half * CHUNK + j * GROUP, GROUP)]],
                        rows.at[pl.ds(j * GROUP, GROUP)], gsem).wait()

            def fire_scatter(rb, half, rows, ssem):
                for j in range(GROUPS_PER_CHUNK):
                    pltpu.async_copy(
                        rows.at[pl.ds(j * GROUP, GROUP)],
                        acc.at[rb.at[pl.ds(half * CHUNK + j * GROUP, GROUP)]],
                        ssem, add=True)

            def drain_scatter(rb, half, rows, ssem):
                for j in range(GROUPS_PER_CHUNK):
                    pltpu.make_async_copy(
                        rows.at[pl.ds(j * GROUP, GROUP)],
                        acc.at[rb.at[pl.ds(half * CHUNK + j * GROUP, GROUP)]],
                        ssem).wait()

            fire_idxpair(0, sbufP, rbufP)
            drain_idxpair(sbufP, rbufP)
            pltpu.sync_copy(zeros_hbm.at[pl.ds(0, CHUNK)], rowsB)
            fire_scatter(rbufP, 0, rowsB, ssemB)
            fire_gather(sbufP, 0, rowsA, gsemA)

            def half(t, P_s, P_r, Q_s, Q_r):
                pair_next = lax.rem(t + 1, BODIES)
                drain_scatter(P_r, 1, rowsB, ssemB)
                fire_idxpair(pair_next, Q_s, Q_r)
                fire_gather(P_s, 1, rowsB, gsemB)
                drain_gather(P_s, 0, rowsA, gsemA)
                fire_scatter(P_r, 0, rowsA, ssemA)
                drain_gather(P_s, 1, rowsB, gsemB)
                fire_scatter(P_r, 1, rowsB, ssemB)
                drain_scatter(P_r, 0, rowsA, ssemA)
                drain_idxpair(Q_s, Q_r)
                fire_gather(Q_s, 0, rowsA, gsemA)

            def body(u, carry):
                half(2 * u, sbufP, rbufP, sbufQ, rbufQ)
                half(2 * u + 1, sbufQ, rbufQ, sbufP, rbufP)
                return carry

            lax.fori_loop(0, BODIES // 2, body, 0)

            drain_gather(sbufP, 0, rowsA, gsemA)
            drain_scatter(rbufP, 1, rowsB, ssemB)

        @pl.when(c == 0)
        def _():
            run(tlo_hbm)

        @pl.when(c == 1)
        def _():
            run(thi_hbm)

        plsc.subcore_barrier()

        @pl.when(s == 0)
        def _():
            pltpu.sync_copy(acc, out_hbm.at[c])

    return scatter_kernel


_scatter = _make_scatter()


def _node_body(lg, hd, parts, w0a, w0b, w0c, w0d, b0, w1, b1, w2l, b2l,
               w2h, b2h, out_lg, out_hd):
    h = jnp.dot(lg[...], w0a[...], preferred_element_type=jnp.float32)
    h += jnp.dot(hd[...], w0b[...], preferred_element_type=jnp.float32)
    h += jnp.dot(parts[0], w0c[...], preferred_element_type=jnp.float32)
    h += jnp.dot(parts[1], w0d[...], preferred_element_type=jnp.float32)
    h = jnp.maximum(h + b0[...], 0.0)
    h = jnp.maximum(jnp.dot(h, w1[...], preferred_element_type=jnp.float32) + b1[...], 0.0)
    out_lg[...] = jnp.dot(h, w2l[...], preferred_element_type=jnp.float32) + b2l[...]
    out_hd[...] = jnp.dot(h, w2h[...], preferred_element_type=jnp.float32) + b2h[...]


def _node_update(node_logits, node_hidden, parts, nW0, nb0, nW1, nb1, nW2, nb2):
    blk = 5000
    grid = N_NODES // blk
    full = lambda shape: pl.BlockSpec(shape, lambda i: tuple(0 for _ in shape))
    w0c = nW0[20:36]
    w0d = jnp.concatenate([nW0[36:40], jnp.zeros((12, 64), jnp.float32)], axis=0)
    return pl.pallas_call(
        _node_body,
        grid=(grid,),
        in_specs=[
            pl.BlockSpec((blk, 4), lambda i: (i, 0)),
            pl.BlockSpec((blk, 16), lambda i: (i, 0)),
            pl.BlockSpec((NC, blk, TW), lambda i: (0, i, 0)),
            full((4, 64)), full((16, 64)), full((TW, 64)), full((TW, 64)),
            full((1, 64)),
            full((64, 32)), full((1, 32)),
            full((32, 4)), full((1, 4)),
            full((32, 16)), full((1, 16)),
        ],
        out_specs=[
            pl.BlockSpec((blk, 4), lambda i: (i, 0)),
            pl.BlockSpec((blk, 16), lambda i: (i, 0)),
        ],
        out_shape=[
            jax.ShapeDtypeStruct((N_NODES, 4), jnp.float32),
            jax.ShapeDtypeStruct((N_NODES, 16), jnp.float32),
        ],
    )(node_logits, node_hidden, parts,
      nW0[:4], nW0[4:20], w0c, w0d,
      nb0.reshape(1, 64),
      nW1, nb1.reshape(1, 32),
      nW2[:, :4], nb2[:4].reshape(1, 4), nW2[:, 4:], nb2[4:].reshape(1, 16))


def kernel(node_logits, node_hidden, senders, receivers,
           edge_W0, edge_b0, edge_W1, edge_b1, edge_W2, edge_b2,
           node_W0, node_b0, node_W1, node_b1, node_W2, node_b2):
    t_lo, t_hi = _edge_table(node_logits, node_hidden,
                             edge_W0, edge_b0, edge_W1, edge_b1,
                             edge_W2, edge_b2)
    zeros = jnp.zeros((N_NODES, TW), jnp.float32)
    parts = _scatter(t_lo, t_hi, senders, receivers, zeros)
    new_logits, new_hidden = _node_update(
        node_logits, node_hidden, parts,
        node_W0, node_b0, node_W1, node_b1, node_W2, node_b2)
    return (new_logits, new_hidden)


# speedup vs baseline: 40.3599x; 1.0198x over previous
import functools

import jax
import jax.numpy as jnp
from jax import lax
from jax.experimental import pallas as pl
from jax.experimental.pallas import tpu as pltpu
from jax.experimental.pallas import tpu_sc as plsc

N_NODES = 100000
N_EDGES = 3200000
TW = 16

NC = 2
NS = 16

GROUP = 80
GROUPS_PER_CHUNK = 5
CHUNK = GROUP * GROUPS_PER_CHUNK
N_CHUNKS = N_EDGES // CHUNK
CHUNKS_PER_TILE = N_CHUNKS // NS
BODIES = CHUNKS_PER_TILE // 2


PACK = 8
NT = 102400
ROWS = NT // PACK


def _bd(w):
    return jnp.kron(jnp.eye(PACK, dtype=jnp.float32), w)


def _tile_b(b):
    return jnp.tile(b, PACK).reshape(1, PACK * b.shape[0])


def _edge_table_body(x, w0, b0, w1, b1, w2l, b2l, w2h, b2h, out_lo, out_hi):
    h = jnp.dot(x[...], w0[...], preferred_element_type=jnp.float32)
    h = jnp.maximum(h + b0[...], 0.0)
    h = jnp.maximum(jnp.dot(h, w1[...], preferred_element_type=jnp.float32) + b1[...], 0.0)
    out_lo[...] = jnp.dot(h, w2l[...], preferred_element_type=jnp.float32) + b2l[...]
    out_hi[...] = jnp.dot(h, w2h[...], preferred_element_type=jnp.float32) + b2h[...]


def _edge_table(feat32p, eW0, eb0, eW1, eb1, eW2, eb2):
    blk = 1600
    grid = ROWS // blk
    full = lambda shape: pl.BlockSpec(shape, lambda i: (0, 0))
    w0 = _bd(jnp.concatenate([eW0, jnp.zeros((12, 64), jnp.float32)], axis=0))
    w2h = jnp.concatenate([eW2[:, 16:20], jnp.zeros((32, 12), jnp.float32)], axis=1)
    b2h = jnp.concatenate([eb2[16:20], jnp.zeros((12,), jnp.float32)])
    return pl.pallas_call(
        _edge_table_body,
        grid=(grid,),
        in_specs=[
            pl.BlockSpec((blk, 32 * PACK), lambda i: (i, 0)),
            full((32 * PACK, 64 * PACK)), full((1, 64 * PACK)),
            full((64 * PACK, 32 * PACK)), full((1, 32 * PACK)),
            full((32 * PACK, TW * PACK)), full((1, TW * PACK)),
            full((32 * PACK, TW * PACK)), full((1, TW * PACK)),
        ],
        out_specs=[
            pl.BlockSpec((blk, TW * PACK), lambda i: (i, 0)),
            pl.BlockSpec((blk, TW * PACK), lambda i: (i, 0)),
        ],
        out_shape=[
            jax.ShapeDtypeStruct((ROWS, TW * PACK), jnp.float32),
            jax.ShapeDtypeStruct((ROWS, TW * PACK), jnp.float32),
        ],
    )(feat32p, w0, _tile_b(eb0),
      _bd(eW1), _tile_b(eb1),
      _bd(eW2[:, :16]), _tile_b(eb2[:16]),
      _bd(w2h), _tile_b(b2h))


def _make_scatter():
    mesh = plsc.VectorSubcoreMesh(core_axis_name="c", subcore_axis_name="s")

    @functools.partial(
        pl.kernel,
        out_type=jax.ShapeDtypeStruct((NC, NT, TW), jnp.float32),
        mesh=mesh,
        scratch_types=[
            pltpu.VMEM((2 * CHUNK,), jnp.int32),
            pltpu.VMEM((2 * CHUNK,), jnp.int32),
            pltpu.VMEM((2 * CHUNK,), jnp.int32),
            pltpu.VMEM((2 * CHUNK,), jnp.int32),
            pltpu.VMEM((CHUNK, TW), jnp.float32),
            pltpu.VMEM((CHUNK, TW), jnp.float32),
            pltpu.VMEM_SHARED((N_NODES, TW), jnp.float32),
            pltpu.SemaphoreType.DMA,
            pltpu.SemaphoreType.DMA,
            pltpu.SemaphoreType.DMA,
            pltpu.SemaphoreType.DMA,
            pltpu.SemaphoreType.DMA,
        ],
        compiler_params=pltpu.CompilerParams(use_tc_tiling_on_sc=False),
    )
    def scatter_kernel(tlo_hbm, thi_hbm, snd_hbm, rcv_hbm, zeros_hbm, out_hbm,
                       sbufP, rbufP, sbufQ, rbufQ, rowsA, rowsB, acc,
                       gsemA, gsemB, ssemA, ssemB, isem):
        c = lax.axis_index("c")
        s = lax.axis_index("s")

        @pl.when(s == 0)
        def _():
            pltpu.sync_copy(zeros_hbm, acc)
        plsc.subcore_barrier()

        def run(tbl):
            def fire_idxpair(pair, sb, rb):
                base = (s * CHUNKS_PER_TILE + 2 * pair) * CHUNK
                pltpu.async_copy(snd_hbm.at[pl.ds(base, 2 * CHUNK)], sb, isem)
                pltpu.async_copy(rcv_hbm.at[pl.ds(base, 2 * CHUNK)], rb, isem)

            def drain_idxpair(sb, rb):
                pltpu.make_async_copy(snd_hbm.at[pl.ds(0, 2 * CHUNK)], sb,
                                      isem).wait()
                pltpu.make_async_copy(rcv_hbm.at[pl.ds(0, 2 * CHUNK)], rb,
                                      isem).wait()

            def fire_gather(sb, half, rows, gsem):
                for j in range(GROUPS_PER_CHUNK):
                    pltpu.async_copy(
                        tbl.at[sb.at[pl.ds(half * CHUNK + j * GROUP, GROUP)]],
                        rows.at[pl.ds(j * GROUP, GROUP)], gsem)

            def drain_gather(sb, half, rows, gsem):
                for j in range(GROUPS_PER_CHUNK):
                    pltpu.make_async_copy(
                        tbl.at[sb.at[pl.ds(half * CHUNK + j * GROUP, GROUP)]],
                        rows.at[pl.ds(j * GROUP, GROUP)], gsem).wait()

            def fire_scatter(rb, half, rows, ssem):
                for j in range(GROUPS_PER_CHUNK):
                    pltpu.async_copy(
                        rows.at[pl.ds(j * GROUP, GROUP)],
                        acc.at[rb.at[pl.ds(half * CHUNK + j * GROUP, GROUP)]],
                        ssem, add=True)

            def drain_scatter(rb, half, rows, ssem):
                for j in range(GROUPS_PER_CHUNK):
                    pltpu.make_async_copy(
                        rows.at[pl.ds(j * GROUP, GROUP)],
                        acc.at[rb.at[pl.ds(half * CHUNK + j * GROUP, GROUP)]],
                        ssem).wait()

            fire_idxpair(0, sbufP, rbufP)
            drain_idxpair(sbufP, rbufP)
            pltpu.sync_copy(zeros_hbm.at[pl.ds(0, CHUNK)], rowsB)
            fire_scatter(rbufP, 0, rowsB, ssemB)
            fire_gather(sbufP, 0, rowsA, gsemA)

            def half(t, P_s, P_r, Q_s, Q_r):
                pair_next = lax.rem(t + 1, BODIES)
                drain_scatter(P_r, 1, rowsB, ssemB)
                fire_idxpair(pair_next, Q_s, Q_r)
                fire_gather(P_s, 1, rowsB, gsemB)
                drain_gather(P_s, 0, rowsA, gsemA)
                fire_scatter(P_r, 0, rowsA, ssemA)
                drain_gather(P_s, 1, rowsB, gsemB)
                fire_scatter(P_r, 1, rowsB, ssemB)
                drain_scatter(P_r, 0, rowsA, ssemA)
                drain_idxpair(Q_s, Q_r)
                fire_gather(Q_s, 0, rowsA, gsemA)

            def body(u, carry):
                half(2 * u, sbufP, rbufP, sbufQ, rbufQ)
                half(2 * u + 1, sbufQ, rbufQ, sbufP, rbufP)
                return carry

            lax.fori_loop(0, BODIES // 2, body, 0)

            drain_gather(sbufP, 0, rowsA, gsemA)
            drain_scatter(rbufP, 1, rowsB, ssemB)

        @pl.when(c == 0)
        def _():
            run(tlo_hbm)

        @pl.when(c == 1)
        def _():
            run(thi_hbm)

        plsc.subcore_barrier()

        @pl.when(s == 0)
        def _():
            pltpu.sync_copy(acc, out_hbm.at[c].at[pl.ds(0, N_NODES)])

    return scatter_kernel


_scatter = _make_scatter()


def _node_body(x, p0, p1, w0f, w0c, w0d, b0, w1, b1, w2, b2, out):
    h = jnp.dot(x[...], w0f[...], preferred_element_type=jnp.float32)
    h += jnp.dot(p0[...], w0c[...], preferred_element_type=jnp.float32)
    h += jnp.dot(p1[...], w0d[...], preferred_element_type=jnp.float32)
    h = jnp.maximum(h + b0[...], 0.0)
    h = jnp.maximum(jnp.dot(h, w1[...], preferred_element_type=jnp.float32) + b1[...], 0.0)
    out[...] = jnp.dot(h, w2[...], preferred_element_type=jnp.float32) + b2[...]


def _node_update(feat32p, p0, p1, nW0, nb0, nW1, nb1, nW2, nb2):
    blk = 1600
    grid = ROWS // blk
    full = lambda shape: pl.BlockSpec(shape, lambda i: (0, 0))
    w0f = _bd(jnp.concatenate([nW0[:20], jnp.zeros((12, 64), jnp.float32)],
                              axis=0))
    w0c = _bd(nW0[20:36])
    w0d = _bd(jnp.concatenate([nW0[36:40], jnp.zeros((12, 64), jnp.float32)],
                              axis=0))
    w2 = _bd(jnp.concatenate([nW2, jnp.zeros((32, 12), jnp.float32)], axis=1))
    b2 = _tile_b(jnp.concatenate([nb2, jnp.zeros((12,), jnp.float32)]))
    return pl.pallas_call(
        _node_body,
        grid=(grid,),
        in_specs=[
            pl.BlockSpec((blk, 32 * PACK), lambda i: (i, 0)),
            pl.BlockSpec((blk, TW * PACK), lambda i: (i, 0)),
            pl.BlockSpec((blk, TW * PACK), lambda i: (i, 0)),
            full((32 * PACK, 64 * PACK)), full((TW * PACK, 64 * PACK)),
            full((TW * PACK, 64 * PACK)), full((1, 64 * PACK)),
            full((64 * PACK, 32 * PACK)), full((1, 32 * PACK)),
            full((32 * PACK, 32 * PACK)), full((1, 32 * PACK)),
        ],
        out_specs=pl.BlockSpec((blk, 32 * PACK), lambda i: (i, 0)),
        out_shape=jax.ShapeDtypeStruct((ROWS, 32 * PACK), jnp.float32),
    )(feat32p, p0, p1,
      w0f, w0c, w0d, _tile_b(nb0),
      _bd(nW1), _tile_b(nb1),
      w2, b2)


def kernel(node_logits, node_hidden, senders, receivers,
           edge_W0, edge_b0, edge_W1, edge_b1, edge_W2, edge_b2,
           node_W0, node_b0, node_W1, node_b1, node_W2, node_b2):
    pad = ((0, NT - N_NODES), (0, 0))
    feat32 = jnp.concatenate(
        [jnp.pad(node_logits, pad), jnp.pad(node_hidden, pad),
         jnp.zeros((NT, 12), jnp.float32)], axis=1)
    feat32p = feat32.reshape(ROWS, 32 * PACK)
    t_lo_p, t_hi_p = _edge_table(feat32p,
                                 edge_W0, edge_b0, edge_W1, edge_b1,
                                 edge_W2, edge_b2)
    zeros = jnp.zeros((N_NODES, TW), jnp.float32)
    parts = _scatter(t_lo_p.reshape(NT, TW), t_hi_p.reshape(NT, TW),
                     senders, receivers, zeros)
    yp = _node_update(feat32p,
                      parts[0].reshape(ROWS, TW * PACK),
                      parts[1].reshape(ROWS, TW * PACK),
                      node_W0, node_b0, node_W1, node_b1, node_W2, node_b2)
    y32 = yp.reshape(NT, 32)
    return (y32[:N_NODES, :4], y32[:N_NODES, 4:20])


# speedup vs baseline: 45.3951x; 1.1248x over previous
import functools

import jax
import jax.numpy as jnp
from jax import lax
from jax.experimental import pallas as pl
from jax.experimental.pallas import tpu as pltpu
from jax.experimental.pallas import tpu_sc as plsc

N_NODES = 100000
N_EDGES = 3200000
TW = 16

NC = 2
NS = 16

GROUP = 80
GROUPS_PER_CHUNK = 5
CHUNK = GROUP * GROUPS_PER_CHUNK
N_CHUNKS = N_EDGES // CHUNK
CHUNKS_PER_TILE = N_CHUNKS // NS
BODIES = CHUNKS_PER_TILE // 2


PACK = 8
NT = 102400
ROWS = NT // PACK


def _bd(w):
    return jnp.kron(jnp.eye(PACK, dtype=jnp.float32), w)


def _tile_b(b):
    return jnp.tile(b, PACK).reshape(1, PACK * b.shape[0])


def _edge_table_body(x, w0, b0, w1, b1, w2l, b2l, w2h, b2h, out_lo, out_hi):
    h = jnp.dot(x[...], w0[...], preferred_element_type=jnp.float32)
    h = jnp.maximum(h + b0[...], 0.0)
    h = jnp.maximum(jnp.dot(h, w1[...], preferred_element_type=jnp.float32) + b1[...], 0.0)
    out_lo[...] = jnp.dot(h, w2l[...], preferred_element_type=jnp.float32) + b2l[...]
    out_hi[...] = jnp.dot(h, w2h[...], preferred_element_type=jnp.float32) + b2h[...]


def _w0_seg(w):
    return jnp.concatenate(
        [_bd(w[:4]), _bd(w[4:20]),
         jnp.zeros((96, 64 * PACK), jnp.float32)], axis=0)


def _edge_table(feat32p, eW0, eb0, eW1, eb1, eW2, eb2):
    blk = 1600
    grid = ROWS // blk
    full = lambda shape: pl.BlockSpec(shape, lambda i: (0, 0))
    w0 = _w0_seg(eW0)
    w2h = jnp.concatenate([eW2[:, 16:20], jnp.zeros((32, 12), jnp.float32)], axis=1)
    b2h = jnp.concatenate([eb2[16:20], jnp.zeros((12,), jnp.float32)])
    return pl.pallas_call(
        _edge_table_body,
        grid=(grid,),
        in_specs=[
            pl.BlockSpec((blk, 32 * PACK), lambda i: (i, 0)),
            full((32 * PACK, 64 * PACK)), full((1, 64 * PACK)),
            full((64 * PACK, 32 * PACK)), full((1, 32 * PACK)),
            full((32 * PACK, TW * PACK)), full((1, TW * PACK)),
            full((32 * PACK, TW * PACK)), full((1, TW * PACK)),
        ],
        out_specs=[
            pl.BlockSpec((blk, TW * PACK), lambda i: (i, 0)),
            pl.BlockSpec((blk, TW * PACK), lambda i: (i, 0)),
        ],
        out_shape=[
            jax.ShapeDtypeStruct((ROWS, TW * PACK), jnp.float32),
            jax.ShapeDtypeStruct((ROWS, TW * PACK), jnp.float32),
        ],
    )(feat32p, w0, _tile_b(eb0),
      _bd(eW1), _tile_b(eb1),
      _bd(eW2[:, :16]), _tile_b(eb2[:16]),
      _bd(w2h), _tile_b(b2h))


def _make_scatter():
    mesh = plsc.VectorSubcoreMesh(core_axis_name="c", subcore_axis_name="s")

    @functools.partial(
        pl.kernel,
        out_type=jax.ShapeDtypeStruct((NC, NT, TW), jnp.float32),
        mesh=mesh,
        scratch_types=[
            pltpu.VMEM((2 * CHUNK,), jnp.int32),
            pltpu.VMEM((2 * CHUNK,), jnp.int32),
            pltpu.VMEM((2 * CHUNK,), jnp.int32),
            pltpu.VMEM((2 * CHUNK,), jnp.int32),
            pltpu.VMEM((CHUNK, TW), jnp.float32),
            pltpu.VMEM((CHUNK, TW), jnp.float32),
            pltpu.VMEM_SHARED((N_NODES, TW), jnp.float32),
            pltpu.SemaphoreType.DMA,
            pltpu.SemaphoreType.DMA,
            pltpu.SemaphoreType.DMA,
            pltpu.SemaphoreType.DMA,
            pltpu.SemaphoreType.DMA,
        ],
        compiler_params=pltpu.CompilerParams(use_tc_tiling_on_sc=False),
    )
    def scatter_kernel(tlo_hbm, thi_hbm, snd_hbm, rcv_hbm, zeros_hbm, out_hbm,
                       sbufP, rbufP, sbufQ, rbufQ, rowsA, rowsB, acc,
                       gsemA, gsemB, ssemA, ssemB, isem):
        c = lax.axis_index("c")
        s = lax.axis_index("s")

        @pl.when(s == 0)
        def _():
            pltpu.sync_copy(zeros_hbm, acc)
        plsc.subcore_barrier()

        def run(tbl):
            def fire_idxpair(pair, sb, rb):
                base = (s * CHUNKS_PER_TILE + 2 * pair) * CHUNK
                pltpu.async_copy(snd_hbm.at[pl.ds(base, 2 * CHUNK)], sb, isem)
                pltpu.async_copy(rcv_hbm.at[pl.ds(base, 2 * CHUNK)], rb, isem)

            def drain_idxpair(sb, rb):
                pltpu.make_async_copy(snd_hbm.at[pl.ds(0, 2 * CHUNK)], sb,
                                      isem).wait()
                pltpu.make_async_copy(rcv_hbm.at[pl.ds(0, 2 * CHUNK)], rb,
                                      isem).wait()

            def fire_gather(sb, half, rows, gsem):
                for j in range(GROUPS_PER_CHUNK):
                    pltpu.async_copy(
                        tbl.at[sb.at[pl.ds(half * CHUNK + j * GROUP, GROUP)]],
                        rows.at[pl.ds(j * GROUP, GROUP)], gsem)

            def drain_gather(sb, half, rows, gsem):
                for j in range(GROUPS_PER_CHUNK):
                    pltpu.make_async_copy(
                        tbl.at[sb.at[pl.ds(half * CHUNK + j * GROUP, GROUP)]],
                        rows.at[pl.ds(j * GROUP, GROUP)], gsem).wait()

            def fire_scatter(rb, half, rows, ssem):
                for j in range(GROUPS_PER_CHUNK):
                    pltpu.async_copy(
                        rows.at[pl.ds(j * GROUP, GROUP)],
                        acc.at[rb.at[pl.ds(half * CHUNK + j * GROUP, GROUP)]],
                        ssem, add=True)

            def drain_scatter(rb, half, rows, ssem):
                for j in range(GROUPS_PER_CHUNK):
                    pltpu.make_async_copy(
                        rows.at[pl.ds(j * GROUP, GROUP)],
                        acc.at[rb.at[pl.ds(half * CHUNK + j * GROUP, GROUP)]],
                        ssem).wait()

            fire_idxpair(0, sbufP, rbufP)
            drain_idxpair(sbufP, rbufP)
            pltpu.sync_copy(zeros_hbm.at[pl.ds(0, CHUNK)], rowsB)
            fire_scatter(rbufP, 0, rowsB, ssemB)
            fire_gather(sbufP, 0, rowsA, gsemA)

            def half(t, P_s, P_r, Q_s, Q_r):
                pair_next = lax.rem(t + 1, BODIES)
                drain_scatter(P_r, 1, rowsB, ssemB)
                fire_idxpair(pair_next, Q_s, Q_r)
                fire_gather(P_s, 1, rowsB, gsemB)
                drain_gather(P_s, 0, rowsA, gsemA)
                fire_scatter(P_r, 0, rowsA, ssemA)
                drain_gather(P_s, 1, rowsB, gsemB)
                fire_scatter(P_r, 1, rowsB, ssemB)
                drain_scatter(P_r, 0, rowsA, ssemA)
                drain_idxpair(Q_s, Q_r)
                fire_gather(Q_s, 0, rowsA, gsemA)

            def body(u, carry):
                half(2 * u, sbufP, rbufP, sbufQ, rbufQ)
                half(2 * u + 1, sbufQ, rbufQ, sbufP, rbufP)
                return carry

            lax.fori_loop(0, BODIES // 2, body, 0)

            drain_gather(sbufP, 0, rowsA, gsemA)
            drain_scatter(rbufP, 1, rowsB, ssemB)

        @pl.when(c == 0)
        def _():
            run(tlo_hbm)

        @pl.when(c == 1)
        def _():
            run(thi_hbm)

        plsc.subcore_barrier()

        @pl.when(s == 0)
        def _():
            pltpu.sync_copy(acc, out_hbm.at[c].at[pl.ds(0, N_NODES)])

    return scatter_kernel


_scatter = _make_scatter()


def _node_body(x, parts, w0f, w0c, w0d, b0, w1, b1, w2l, b2l, w2h, b2h,
               out_lg, out_hd):
    p = parts[...]
    h = jnp.dot(x[...], w0f[...], preferred_element_type=jnp.float32)
    h += jnp.dot(p[0], w0c[...], preferred_element_type=jnp.float32)
    h += jnp.dot(p[1], w0d[...], preferred_element_type=jnp.float32)
    h = jnp.maximum(h + b0[...], 0.0)
    h = jnp.maximum(jnp.dot(h, w1[...], preferred_element_type=jnp.float32) + b1[...], 0.0)
    out_lg[...] = jnp.dot(h, w2l[...], preferred_element_type=jnp.float32) + b2l[...]
    out_hd[...] = jnp.dot(h, w2h[...], preferred_element_type=jnp.float32) + b2h[...]


def _node_update(feat32p, parts, nW0, nb0, nW1, nb1, nW2, nb2):
    blk = 1600
    grid = ROWS // blk
    full = lambda shape: pl.BlockSpec(shape, lambda i: tuple(0 for _ in shape))
    w0f = _w0_seg(nW0[:20])
    w0c = _bd(nW0[20:36])
    w0d = _bd(jnp.concatenate([nW0[36:40], jnp.zeros((12, 64), jnp.float32)],
                              axis=0))
    return pl.pallas_call(
        _node_body,
        grid=(grid,),
        in_specs=[
            pl.BlockSpec((blk, 32 * PACK), lambda i: (i, 0)),
            pl.BlockSpec((NC, blk, TW * PACK), lambda i: (0, i, 0)),
            full((32 * PACK, 64 * PACK)), full((TW * PACK, 64 * PACK)),
            full((TW * PACK, 64 * PACK)), full((1, 64 * PACK)),
            full((64 * PACK, 32 * PACK)), full((1, 32 * PACK)),
            full((32 * PACK, 4 * PACK)), full((1, 4 * PACK)),
            full((32 * PACK, TW * PACK)), full((1, TW * PACK)),
        ],
        out_specs=[
            pl.BlockSpec((blk, 4 * PACK), lambda i: (i, 0)),
            pl.BlockSpec((blk, TW * PACK), lambda i: (i, 0)),
        ],
        out_shape=[
            jax.ShapeDtypeStruct((ROWS, 4 * PACK), jnp.float32),
            jax.ShapeDtypeStruct((ROWS, TW * PACK), jnp.float32),
        ],
    )(feat32p, parts,
      w0f, w0c, w0d, _tile_b(nb0),
      _bd(nW1), _tile_b(nb1),
      _bd(nW2[:, :4]), _tile_b(nb2[:4]),
      _bd(nW2[:, 4:20]), _tile_b(nb2[4:20]))


def kernel(node_logits, node_hidden, senders, receivers,
           edge_W0, edge_b0, edge_W1, edge_b1, edge_W2, edge_b2,
           node_W0, node_b0, node_W1, node_b1, node_W2, node_b2):
    pad = ((0, NT - N_NODES), (0, 0))
    feat32p = jnp.concatenate(
        [jnp.pad(node_logits, pad).reshape(ROWS, 4 * PACK),
         jnp.pad(node_hidden, pad).reshape(ROWS, TW * PACK),
         jnp.zeros((ROWS, 96), jnp.float32)], axis=1)
    t_lo_p, t_hi_p = _edge_table(feat32p,
                                 edge_W0, edge_b0, edge_W1, edge_b1,
                                 edge_W2, edge_b2)
    zeros = jnp.zeros((N_NODES, TW), jnp.float32)
    parts = _scatter(t_lo_p.reshape(NT, TW), t_hi_p.reshape(NT, TW),
                     senders, receivers, zeros)
    out_lgp, out_hdp = _node_update(
        feat32p, parts.reshape(NC, ROWS, TW * PACK),
        node_W0, node_b0, node_W1, node_b1, node_W2, node_b2)
    return (out_lgp.reshape(NT, 4)[:N_NODES],
            out_hdp.reshape(NT, TW)[:N_NODES])


# speedup vs baseline: 48.6536x; 1.0718x over previous
import functools

import jax
import jax.numpy as jnp
from jax import lax
from jax.experimental import pallas as pl
from jax.experimental.pallas import tpu as pltpu
from jax.experimental.pallas import tpu_sc as plsc

N_NODES = 100000
N_EDGES = 3200000
TW = 16

NC = 2
NS = 16

GROUP = 80
GROUPS_PER_CHUNK = 5
CHUNK = GROUP * GROUPS_PER_CHUNK
N_CHUNKS = N_EDGES // CHUNK
CHUNKS_PER_TILE = N_CHUNKS // NS
BODIES = CHUNKS_PER_TILE // 2


PACK = 8
NT = 102400
ROWS = NT // PACK


def _bd(w):
    return jnp.kron(jnp.eye(PACK, dtype=jnp.float32), w)


def _tile_b(b):
    return jnp.tile(b, PACK).reshape(1, PACK * b.shape[0])


def _edge_table_body(lgp, hdp, w0a, w0b, b0, w1, b1, w2l, b2l, w2h, b2h,
                     out_lo, out_hi):
    h = jnp.dot(lgp[...], w0a[...], preferred_element_type=jnp.float32)
    h += jnp.dot(hdp[...], w0b[...], preferred_element_type=jnp.float32)
    h = jnp.maximum(h + b0[...], 0.0)
    h = jnp.maximum(jnp.dot(h, w1[...], preferred_element_type=jnp.float32) + b1[...], 0.0)
    out_lo[...] = jnp.dot(h, w2l[...], preferred_element_type=jnp.float32) + b2l[...]
    out_hi[...] = jnp.dot(h, w2h[...], preferred_element_type=jnp.float32) + b2h[...]


def _edge_table(lgp, hdp, eW0, eb0, eW1, eb1, eW2, eb2):
    blk = 1600
    grid = ROWS // blk
    full = lambda shape: pl.BlockSpec(shape, lambda i: (0, 0))
    w2h = jnp.concatenate([eW2[:, 16:20], jnp.zeros((32, 12), jnp.float32)], axis=1)
    b2h = jnp.concatenate([eb2[16:20], jnp.zeros((12,), jnp.float32)])
    return pl.pallas_call(
        _edge_table_body,
        grid=(grid,),
        in_specs=[
            pl.BlockSpec((blk, 4 * PACK), lambda i: (i, 0)),
            pl.BlockSpec((blk, TW * PACK), lambda i: (i, 0)),
            full((4 * PACK, 64 * PACK)), full((TW * PACK, 64 * PACK)),
            full((1, 64 * PACK)),
            full((64 * PACK, 32 * PACK)), full((1, 32 * PACK)),
            full((32 * PACK, TW * PACK)), full((1, TW * PACK)),
            full((32 * PACK, TW * PACK)), full((1, TW * PACK)),
        ],
        out_specs=[
            pl.BlockSpec((blk, TW * PACK), lambda i: (i, 0)),
            pl.BlockSpec((blk, TW * PACK), lambda i: (i, 0)),
        ],
        out_shape=[
            jax.ShapeDtypeStruct((ROWS, TW * PACK), jnp.float32),
            jax.ShapeDtypeStruct((ROWS, TW * PACK), jnp.float32),
        ],
    )(lgp, hdp, _bd(eW0[:4]), _bd(eW0[4:20]), _tile_b(eb0),
      _bd(eW1), _tile_b(eb1),
      _bd(eW2[:, :16]), _tile_b(eb2[:16]),
      _bd(w2h), _tile_b(b2h))


def _make_scatter():
    mesh = plsc.VectorSubcoreMesh(core_axis_name="c", subcore_axis_name="s")

    @functools.partial(
        pl.kernel,
        out_type=jax.ShapeDtypeStruct((NC, NT, TW), jnp.float32),
        mesh=mesh,
        scratch_types=[
            pltpu.VMEM((2 * CHUNK,), jnp.int32),
            pltpu.VMEM((2 * CHUNK,), jnp.int32),
            pltpu.VMEM((2 * CHUNK,), jnp.int32),
            pltpu.VMEM((2 * CHUNK,), jnp.int32),
            pltpu.VMEM((CHUNK, TW), jnp.float32),
            pltpu.VMEM((CHUNK, TW), jnp.float32),
            pltpu.VMEM_SHARED((N_NODES, TW), jnp.float32),
            pltpu.SemaphoreType.DMA,
            pltpu.SemaphoreType.DMA,
            pltpu.SemaphoreType.DMA,
            pltpu.SemaphoreType.DMA,
            pltpu.SemaphoreType.DMA,
        ],
        compiler_params=pltpu.CompilerParams(use_tc_tiling_on_sc=False),
    )
    def scatter_kernel(tlo_hbm, thi_hbm, snd_hbm, rcv_hbm, zeros_hbm, out_hbm,
                       sbufP, rbufP, sbufQ, rbufQ, rowsA, rowsB, acc,
                       gsemA, gsemB, ssemA, ssemB, isem):
        c = lax.axis_index("c")
        s = lax.axis_index("s")

        @pl.when(s == 0)
        def _():
            pltpu.sync_copy(zeros_hbm, acc)
        plsc.subcore_barrier()

        def run(tbl):
            def fire_idxpair(pair, sb, rb):
                base = (s * CHUNKS_PER_TILE + 2 * pair) * CHUNK
                pltpu.async_copy(snd_hbm.at[pl.ds(base, 2 * CHUNK)], sb, isem)
                pltpu.async_copy(rcv_hbm.at[pl.ds(base, 2 * CHUNK)], rb, isem)

            def drain_idxpair(sb, rb):
                pltpu.make_async_copy(snd_hbm.at[pl.ds(0, 2 * CHUNK)], sb,
                                      isem).wait()
                pltpu.make_async_copy(rcv_hbm.at[pl.ds(0, 2 * CHUNK)], rb,
                                      isem).wait()

            def fire_gather(sb, half, rows, gsem):
                for j in range(GROUPS_PER_CHUNK):
                    pltpu.async_copy(
                        tbl.at[sb.at[pl.ds(half * CHUNK + j * GROUP, GROUP)]],
                        rows.at[pl.ds(j * GROUP, GROUP)], gsem)

            def drain_gather(sb, half, rows, gsem):
                for j in range(GROUPS_PER_CHUNK):
                    pltpu.make_async_copy(
                        tbl.at[sb.at[pl.ds(half * CHUNK + j * GROUP, GROUP)]],
                        rows.at[pl.ds(j * GROUP, GROUP)], gsem).wait()

            def fire_scatter(rb, half, rows, ssem):
                for j in range(GROUPS_PER_CHUNK):
                    pltpu.async_copy(
                        rows.at[pl.ds(j * GROUP, GROUP)],
                        acc.at[rb.at[pl.ds(half * CHUNK + j * GROUP, GROUP)]],
                        ssem, add=True)

            def drain_scatter(rb, half, rows, ssem):
                for j in range(GROUPS_PER_CHUNK):
                    pltpu.make_async_copy(
                        rows.at[pl.ds(j * GROUP, GROUP)],
                        acc.at[rb.at[pl.ds(half * CHUNK + j * GROUP, GROUP)]],
                        ssem).wait()

            fire_idxpair(0, sbufP, rbufP)
            drain_idxpair(sbufP, rbufP)
            pltpu.sync_copy(zeros_hbm.at[pl.ds(0, CHUNK)], rowsB)
            fire_scatter(rbufP, 0, rowsB, ssemB)
            fire_gather(sbufP, 0, rowsA, gsemA)

            def half(t, P_s, P_r, Q_s, Q_r):
                pair_next = lax.rem(t + 1, BODIES)
                drain_scatter(P_r, 1, rowsB, ssemB)
                fire_idxpair(pair_next, Q_s, Q_r)
                fire_gather(P_s, 1, rowsB, gsemB)
                drain_gather(P_s, 0, rowsA, gsemA)
                fire_scatter(P_r, 0, rowsA, ssemA)
                drain_gather(P_s, 1, rowsB, gsemB)
                fire_scatter(P_r, 1, rowsB, ssemB)
                drain_scatter(P_r, 0, rowsA, ssemA)
                drain_idxpair(Q_s, Q_r)
                fire_gather(Q_s, 0, rowsA, gsemA)

            def body(u, carry):
                half(2 * u, sbufP, rbufP, sbufQ, rbufQ)
                half(2 * u + 1, sbufQ, rbufQ, sbufP, rbufP)
                return carry

            lax.fori_loop(0, BODIES // 2, body, 0)

            drain_gather(sbufP, 0, rowsA, gsemA)
            drain_scatter(rbufP, 1, rowsB, ssemB)

        @pl.when(c == 0)
        def _():
            run(tlo_hbm)

        @pl.when(c == 1)
        def _():
            run(thi_hbm)

        plsc.subcore_barrier()

        @pl.when(s == 0)
        def _():
            pltpu.sync_copy(acc, out_hbm.at[c].at[pl.ds(0, N_NODES)])

    return scatter_kernel


_scatter = _make_scatter()


def _node_body(lgp, hdp, parts, w0fa, w0fb, w0c, w0d, b0, w1, b1, w2l, b2l,
               w2h, b2h, out_lg, out_hd):
    p = parts[...]
    h = jnp.dot(lgp[...], w0fa[...], preferred_element_type=jnp.float32)
    h += jnp.dot(hdp[...], w0fb[...], preferred_element_type=jnp.float32)
    h += jnp.dot(p[0], w0c[...], preferred_element_type=jnp.float32)
    h += jnp.dot(p[1], w0d[...], preferred_element_type=jnp.float32)
    h = jnp.maximum(h + b0[...], 0.0)
    h = jnp.maximum(jnp.dot(h, w1[...], preferred_element_type=jnp.float32) + b1[...], 0.0)
    out_lg[...] = jnp.dot(h, w2l[...], preferred_element_type=jnp.float32) + b2l[...]
    out_hd[...] = jnp.dot(h, w2h[...], preferred_element_type=jnp.float32) + b2h[...]


def _node_update(lgp, hdp, parts, nW0, nb0, nW1, nb1, nW2, nb2):
    blk = 1600
    grid = ROWS // blk
    full = lambda shape: pl.BlockSpec(shape, lambda i: tuple(0 for _ in shape))
    w0c = _bd(nW0[20:36])
    w0d = _bd(jnp.concatenate([nW0[36:40], jnp.zeros((12, 64), jnp.float32)],
                              axis=0))
    return pl.pallas_call(
        _node_body,
        grid=(grid,),
        in_specs=[
            pl.BlockSpec((blk, 4 * PACK), lambda i: (i, 0)),
            pl.BlockSpec((blk, TW * PACK), lambda i: (i, 0)),
            pl.BlockSpec((NC, blk, TW * PACK), lambda i: (0, i, 0)),
            full((4 * PACK, 64 * PACK)), full((TW * PACK, 64 * PACK)),
            full((TW * PACK, 64 * PACK)),
            full((TW * PACK, 64 * PACK)), full((1, 64 * PACK)),
            full((64 * PACK, 32 * PACK)), full((1, 32 * PACK)),
            full((32 * PACK, 4 * PACK)), full((1, 4 * PACK)),
            full((32 * PACK, TW * PACK)), full((1, TW * PACK)),
        ],
        out_specs=[
            pl.BlockSpec((blk, 4 * PACK), lambda i: (i, 0)),
            pl.BlockSpec((blk, TW * PACK), lambda i: (i, 0)),
        ],
        out_shape=[
            jax.ShapeDtypeStruct((ROWS, 4 * PACK), jnp.float32),
            jax.ShapeDtypeStruct((ROWS, TW * PACK), jnp.float32),
        ],
    )(lgp, hdp, parts,
      _bd(nW0[:4]), _bd(nW0[4:20]), w0c, w0d, _tile_b(nb0),
      _bd(nW1), _tile_b(nb1),
      _bd(nW2[:, :4]), _tile_b(nb2[:4]),
      _bd(nW2[:, 4:20]), _tile_b(nb2[4:20]))


def kernel(node_logits, node_hidden, senders, receivers,
           edge_W0, edge_b0, edge_W1, edge_b1, edge_W2, edge_b2,
           node_W0, node_b0, node_W1, node_b1, node_W2, node_b2):
    lgp = jnp.pad(node_logits.reshape(-1),
                  (0, (NT - N_NODES) * 4)).reshape(ROWS, 4 * PACK)
    hdp = jnp.pad(node_hidden.reshape(-1),
                  (0, (NT - N_NODES) * TW)).reshape(ROWS, TW * PACK)
    t_lo_p, t_hi_p = _edge_table(lgp, hdp,
                                 edge_W0, edge_b0, edge_W1, edge_b1,
                                 edge_W2, edge_b2)
    zeros = jnp.zeros((N_NODES, TW), jnp.float32)
    parts = _scatter(t_lo_p.reshape(NT, TW), t_hi_p.reshape(NT, TW),
                     senders, receivers, zeros)
    out_lgp, out_hdp = _node_update(
        lgp, hdp, parts.reshape(NC, ROWS, TW * PACK),
        node_W0, node_b0, node_W1, node_b1, node_W2, node_b2)
    return (out_lgp.reshape(-1)[:N_NODES * 4].reshape(N_NODES, 4),
            out_hdp.reshape(-1)[:N_NODES * TW].reshape(N_NODES, TW))


# speedup vs baseline: 49.0500x; 1.0081x over previous
import functools

import jax
import jax.numpy as jnp
from jax import lax
from jax.experimental import pallas as pl
from jax.experimental.pallas import tpu as pltpu
from jax.experimental.pallas import tpu_sc as plsc

N_NODES = 100000
N_EDGES = 3200000
TW = 16

NC = 2
NS = 16

GROUP = 80
GROUPS_PER_CHUNK = 5
CHUNK = GROUP * GROUPS_PER_CHUNK
N_CHUNKS = N_EDGES // CHUNK
CHUNKS_PER_TILE = N_CHUNKS // NS
BODIES = CHUNKS_PER_TILE // 2


PACK = 8
NT = 102400
ROWS = NT // PACK


def _bd(w):
    return jnp.kron(jnp.eye(PACK, dtype=jnp.float32), w)


def _tile_b(b):
    return jnp.tile(b, PACK).reshape(1, PACK * b.shape[0])


def _edge_table_body(lgp, hdp, w0a, w0b, b0, w1, b1, w2l, b2l, w2h, b2h,
                     out_lo, out_hi):
    h = jnp.dot(lgp[...], w0a[...], preferred_element_type=jnp.float32)
    h += jnp.dot(hdp[...], w0b[...], preferred_element_type=jnp.float32)
    h = jnp.maximum(h + b0[...], 0.0)
    h = jnp.maximum(jnp.dot(h, w1[...], preferred_element_type=jnp.float32) + b1[...], 0.0)
    out_lo[...] = jnp.dot(h, w2l[...], preferred_element_type=jnp.float32) + b2l[...]
    out_hi[...] = jnp.dot(h, w2h[...], preferred_element_type=jnp.float32) + b2h[...]


def _edge_table(lgp, hdp, eW0, eb0, eW1, eb1, eW2, eb2):
    blk = 3200
    grid = ROWS // blk
    full = lambda shape: pl.BlockSpec(shape, lambda i: (0, 0))
    w2h = jnp.concatenate([eW2[:, 16:20], jnp.zeros((32, 12), jnp.float32)], axis=1)
    b2h = jnp.concatenate([eb2[16:20], jnp.zeros((12,), jnp.float32)])
    return pl.pallas_call(
        _edge_table_body,
        grid=(grid,),
        in_specs=[
            pl.BlockSpec((blk, 4 * PACK), lambda i: (i, 0)),
            pl.BlockSpec((blk, TW * PACK), lambda i: (i, 0)),
            full((4 * PACK, 64 * PACK)), full((TW * PACK, 64 * PACK)),
            full((1, 64 * PACK)),
            full((64 * PACK, 32 * PACK)), full((1, 32 * PACK)),
            full((32 * PACK, TW * PACK)), full((1, TW * PACK)),
            full((32 * PACK, TW * PACK)), full((1, TW * PACK)),
        ],
        out_specs=[
            pl.BlockSpec((blk, TW * PACK), lambda i: (i, 0)),
            pl.BlockSpec((blk, TW * PACK), lambda i: (i, 0)),
        ],
        out_shape=[
            jax.ShapeDtypeStruct((ROWS, TW * PACK), jnp.float32),
            jax.ShapeDtypeStruct((ROWS, TW * PACK), jnp.float32),
        ],
    )(lgp, hdp, _bd(eW0[:4]), _bd(eW0[4:20]), _tile_b(eb0),
      _bd(eW1), _tile_b(eb1),
      _bd(eW2[:, :16]), _tile_b(eb2[:16]),
      _bd(w2h), _tile_b(b2h))


def _make_scatter():
    mesh = plsc.VectorSubcoreMesh(core_axis_name="c", subcore_axis_name="s")

    @functools.partial(
        pl.kernel,
        out_type=jax.ShapeDtypeStruct((NC, NT, TW), jnp.float32),
        mesh=mesh,
        scratch_types=[
            pltpu.VMEM((2 * CHUNK,), jnp.int32),
            pltpu.VMEM((2 * CHUNK,), jnp.int32),
            pltpu.VMEM((2 * CHUNK,), jnp.int32),
            pltpu.VMEM((2 * CHUNK,), jnp.int32),
            pltpu.VMEM((CHUNK, TW), jnp.float32),
            pltpu.VMEM((CHUNK, TW), jnp.float32),
            pltpu.VMEM_SHARED((N_NODES, TW), jnp.float32),
            pltpu.SemaphoreType.DMA,
            pltpu.SemaphoreType.DMA,
            pltpu.SemaphoreType.DMA,
            pltpu.SemaphoreType.DMA,
            pltpu.SemaphoreType.DMA,
        ],
        compiler_params=pltpu.CompilerParams(use_tc_tiling_on_sc=False),
    )
    def scatter_kernel(tlo_hbm, thi_hbm, snd_hbm, rcv_hbm, out_hbm,
                       sbufP, rbufP, sbufQ, rbufQ, rowsA, rowsB, acc,
                       gsemA, gsemB, ssemA, ssemB, isem):
        c = lax.axis_index("c")
        s = lax.axis_index("s")

        def zrow(i, carry):
            rowsB[i] = jnp.zeros((TW,), jnp.float32)
            return carry
        lax.fori_loop(0, CHUNK, zrow, 0)
        row0 = s * (N_NODES // NS)
        for m in range(15):
            pltpu.sync_copy(rowsB, acc.at[pl.ds(row0 + m * CHUNK, CHUNK)])
        pltpu.sync_copy(rowsB.at[pl.ds(0, 250)],
                        acc.at[pl.ds(row0 + 15 * CHUNK, 250)])
        plsc.subcore_barrier()

        def run(tbl):
            def fire_idxpair(pair, sb, rb):
                base = (s * CHUNKS_PER_TILE + 2 * pair) * CHUNK
                pltpu.async_copy(snd_hbm.at[pl.ds(base, 2 * CHUNK)], sb, isem)
                pltpu.async_copy(rcv_hbm.at[pl.ds(base, 2 * CHUNK)], rb, isem)

            def drain_idxpair(sb, rb):
                pltpu.make_async_copy(snd_hbm.at[pl.ds(0, 2 * CHUNK)], sb,
                                      isem).wait()
                pltpu.make_async_copy(rcv_hbm.at[pl.ds(0, 2 * CHUNK)], rb,
                                      isem).wait()

            def fire_gather(sb, half, rows, gsem):
                for j in range(GROUPS_PER_CHUNK):
                    pltpu.async_copy(
                        tbl.at[sb.at[pl.ds(half * CHUNK + j * GROUP, GROUP)]],
                        rows.at[pl.ds(j * GROUP, GROUP)], gsem)

            def drain_gather(sb, half, rows, gsem):
                for j in range(GROUPS_PER_CHUNK):
                    pltpu.make_async_copy(
                        tbl.at[sb.at[pl.ds(half * CHUNK + j * GROUP, GROUP)]],
                        rows.at[pl.ds(j * GROUP, GROUP)], gsem).wait()

            def fire_scatter(rb, half, rows, ssem):
                for j in range(GROUPS_PER_CHUNK):
                    pltpu.async_copy(
                        rows.at[pl.ds(j * GROUP, GROUP)],
                        acc.at[rb.at[pl.ds(half * CHUNK + j * GROUP, GROUP)]],
                        ssem, add=True)

            def drain_scatter(rb, half, rows, ssem):
                for j in range(GROUPS_PER_CHUNK):
                    pltpu.make_async_copy(
                        rows.at[pl.ds(j * GROUP, GROUP)],
                        acc.at[rb.at[pl.ds(half * CHUNK + j * GROUP, GROUP)]],
                        ssem).wait()

            fire_idxpair(0, sbufP, rbufP)
            drain_idxpair(sbufP, rbufP)
            fire_scatter(rbufP, 0, rowsB, ssemB)
            fire_gather(sbufP, 0, rowsA, gsemA)

            def half(t, P_s, P_r, Q_s, Q_r):
                pair_next = lax.rem(t + 1, BODIES)
                drain_scatter(P_r, 1, rowsB, ssemB)
                fire_idxpair(pair_next, Q_s, Q_r)
                fire_gather(P_s, 1, rowsB, gsemB)
                drain_gather(P_s, 0, rowsA, gsemA)
                fire_scatter(P_r, 0, rowsA, ssemA)
                drain_gather(P_s, 1, rowsB, gsemB)
                fire_scatter(P_r, 1, rowsB, ssemB)
                drain_scatter(P_r, 0, rowsA, ssemA)
                drain_idxpair(Q_s, Q_r)
                fire_gather(Q_s, 0, rowsA, gsemA)

            def body(u, carry):
                half(2 * u, sbufP, rbufP, sbufQ, rbufQ)
                half(2 * u + 1, sbufQ, rbufQ, sbufP, rbufP)
                return carry

            lax.fori_loop(0, BODIES // 2, body, 0)

            drain_gather(sbufP, 0, rowsA, gsemA)
            drain_scatter(rbufP, 1, rowsB, ssemB)

        @pl.when(c == 0)
        def _():
            run(tlo_hbm)

        @pl.when(c == 1)
        def _():
            run(thi_hbm)

        plsc.subcore_barrier()

        @pl.when(s == 0)
        def _():
            pltpu.sync_copy(acc, out_hbm.at[c].at[pl.ds(0, N_NODES)])

    return scatter_kernel


_scatter = _make_scatter()


def _node_body(lgp, hdp, parts, w0fa, w0fb, w0c, w0d, b0, w1, b1, w2l, b2l,
               w2h, b2h, out_lg, out_hd):
    p = parts[...]
    h = jnp.dot(lgp[...], w0fa[...], preferred_element_type=jnp.float32)
    h += jnp.dot(hdp[...], w0fb[...], preferred_element_type=jnp.float32)
    h += jnp.dot(p[0], w0c[...], preferred_element_type=jnp.float32)
    h += jnp.dot(p[1], w0d[...], preferred_element_type=jnp.float32)
    h = jnp.maximum(h + b0[...], 0.0)
    h = jnp.maximum(jnp.dot(h, w1[...], preferred_element_type=jnp.float32) + b1[...], 0.0)
    out_lg[...] = jnp.dot(h, w2l[...], preferred_element_type=jnp.float32) + b2l[...]
    out_hd[...] = jnp.dot(h, w2h[...], preferred_element_type=jnp.float32) + b2h[...]


def _node_update(lgp, hdp, parts, nW0, nb0, nW1, nb1, nW2, nb2):
    blk = 3200
    grid = ROWS // blk
    full = lambda shape: pl.BlockSpec(shape, lambda i: tuple(0 for _ in shape))
    w0c = _bd(nW0[20:36])
    w0d = _bd(jnp.concatenate([nW0[36:40], jnp.zeros((12, 64), jnp.float32)],
                              axis=0))
    return pl.pallas_call(
        _node_body,
        grid=(grid,),
        in_specs=[
            pl.BlockSpec((blk, 4 * PACK), lambda i: (i, 0)),
            pl.BlockSpec((blk, TW * PACK), lambda i: (i, 0)),
            pl.BlockSpec((NC, blk, TW * PACK), lambda i: (0, i, 0)),
            full((4 * PACK, 64 * PACK)), full((TW * PACK, 64 * PACK)),
            full((TW * PACK, 64 * PACK)),
            full((TW * PACK, 64 * PACK)), full((1, 64 * PACK)),
            full((64 * PACK, 32 * PACK)), full((1, 32 * PACK)),
            full((32 * PACK, 4 * PACK)), full((1, 4 * PACK)),
            full((32 * PACK, TW * PACK)), full((1, TW * PACK)),
        ],
        out_specs=[
            pl.BlockSpec((blk, 4 * PACK), lambda i: (i, 0)),
            pl.BlockSpec((blk, TW * PACK), lambda i: (i, 0)),
        ],
        out_shape=[
            jax.ShapeDtypeStruct((ROWS, 4 * PACK), jnp.float32),
            jax.ShapeDtypeStruct((ROWS, TW * PACK), jnp.float32),
        ],
    )(lgp, hdp, parts,
      _bd(nW0[:4]), _bd(nW0[4:20]), w0c, w0d, _tile_b(nb0),
      _bd(nW1), _tile_b(nb1),
      _bd(nW2[:, :4]), _tile_b(nb2[:4]),
      _bd(nW2[:, 4:20]), _tile_b(nb2[4:20]))


def kernel(node_logits, node_hidden, senders, receivers,
           edge_W0, edge_b0, edge_W1, edge_b1, edge_W2, edge_b2,
           node_W0, node_b0, node_W1, node_b1, node_W2, node_b2):
    lgp = jnp.pad(node_logits.reshape(-1),
                  (0, (NT - N_NODES) * 4)).reshape(ROWS, 4 * PACK)
    hdp = jnp.pad(node_hidden.reshape(-1),
                  (0, (NT - N_NODES) * TW)).reshape(ROWS, TW * PACK)
    t_lo_p, t_hi_p = _edge_table(lgp, hdp,
                                 edge_W0, edge_b0, edge_W1, edge_b1,
                                 edge_W2, edge_b2)
    parts = _scatter(t_lo_p.reshape(NT, TW), t_hi_p.reshape(NT, TW),
                     senders, receivers)
    out_lgp, out_hdp = _node_update(
        lgp, hdp, parts.reshape(NC, ROWS, TW * PACK),
        node_W0, node_b0, node_W1, node_b1, node_W2, node_b2)
    return (out_lgp.reshape(-1)[:N_NODES * 4].reshape(N_NODES, 4),
            out_hdp.reshape(-1)[:N_NODES * TW].reshape(N_NODES, TW))
